# baseline (speedup 1.0000x reference)
import functools

import jax
import jax.numpy as jnp
from jax import lax
from jax.experimental import pallas as pl
from jax.experimental.pallas import tpu as pltpu
from jax.experimental.pallas import tpu_sc as plsc

_DELAYS = (1, 2, 4, 8, 16, 32)
_CHUNK = 256


def kernel(embeddings):
    B, S, D = embeddings.shape
    K = len(_DELAYS)

    items = []
    for b in range(B):
        for k, d in enumerate(_DELAYS):
            items.append((0, 0, d, b, k))
            for c in range(S // _CHUNK):
                dst0 = max(c * _CHUNK, d)
                n = (c + 1) * _CHUNK - dst0
                items.append((dst0 - d, dst0, n, b, k))

    info = plsc.get_sparse_core_info()
    nw = info.num_cores * info.num_subcores
    mesh = plsc.VectorSubcoreMesh(core_axis_name="c", subcore_axis_name="s")

    @functools.partial(
        pl.kernel,
        out_type=jax.ShapeDtypeStruct((B, S, K * D), jnp.float32),
        mesh=mesh,
        compiler_params=pltpu.CompilerParams(use_tc_tiling_on_sc=False),
    )
    def run(emb_hbm, out_hbm):
        wid = lax.axis_index("s") * info.num_cores + lax.axis_index("c")
        for i, (s0, t0, n, b, k) in enumerate(items):
            @pl.when(wid == (i % nw))
            def _copy(s0=s0, t0=t0, n=n, b=b, k=k):
                pltpu.sync_copy(
                    emb_hbm.at[b, pl.ds(s0, n), :],
                    out_hbm.at[b, pl.ds(t0, n), pl.ds(k * D, D)],
                )

    return run(embeddings)


# speedup vs baseline: 48.9565x; 48.9565x over previous
import functools

import jax
import jax.numpy as jnp
from jax import lax
from jax.experimental import pallas as pl
from jax.experimental.pallas import tpu as pltpu
from jax.experimental.pallas import tpu_sc as plsc

_SC_DELAYS = ((3, 8), (4, 16), (5, 32))
_TC_DELAYS = ((0, 1), (1, 2), (2, 4))
_K = 6
_HALO = 32
_SC_CHUNK = 64
_TC_BLOCK = 256


def _sc_part(embeddings):
    B, S, D = embeddings.shape
    C = _SC_CHUNK

    info = plsc.get_sparse_core_info()
    nw = info.num_cores * info.num_subcores
    assert S // C == nw
    mesh = plsc.VectorSubcoreMesh(core_axis_name="c", subcore_axis_name="s")

    @functools.partial(
        pl.kernel,
        out_type=jax.ShapeDtypeStruct((B, S, _K * D), jnp.float32),
        mesh=mesh,
        scratch_types=[
            pltpu.VMEM((_HALO + C, D), jnp.float32),
            pltpu.SemaphoreType.DMA,
        ],
    )
    def run(emb_hbm, out_hbm, buf, sem):
        wid = lax.axis_index("s") * info.num_cores + lax.axis_index("c")
        t0 = wid * C

        def copies(b, do):
            for k, d in _SC_DELAYS:
                ksl = pl.ds(k * D, D)

                @pl.when(wid > 0)
                def _body(k=k, d=d, ksl=ksl):
                    do(buf.at[pl.ds(_HALO - d, C)],
                       out_hbm.at[b, pl.ds(t0, C), ksl], sem)

                @pl.when(wid == 0)
                def _first(k=k, d=d, ksl=ksl):
                    do(buf.at[pl.ds(0, d)],
                       out_hbm.at[b, pl.ds(0, d), ksl], sem)
                    do(buf.at[pl.ds(0, C - d)],
                       out_hbm.at[b, pl.ds(d, C - d), ksl], sem)

        def issue(src, dst, sem):
            pltpu.async_copy(src, dst, sem)

        def drain(src, dst, sem):
            pltpu.make_async_copy(src, dst, sem).wait()

        for b in range(B):
            @pl.when(wid > 0)
            def _stage(b=b):
                pltpu.sync_copy(
                    emb_hbm.at[b, pl.ds(t0 - _HALO, _HALO + C), :], buf)

            @pl.when(wid == 0)
            def _stage0(b=b):
                pltpu.sync_copy(
                    emb_hbm.at[b, pl.ds(0, C), :], buf.at[pl.ds(0, C)])

            copies(b, issue)
            copies(b, drain)

    return run(embeddings)


def _tc_kernel(emb_ref, prev_ref, out_sc_ref, out_ref):
    del out_sc_ref
    i = pl.program_id(1)
    T = _TC_BLOCK
    cur = emb_ref[0]
    prev = prev_ref[0]
    row = lax.broadcasted_iota(jnp.int32, (T, 1), 0)
    for k, d in _TC_DELAYS:
        shifted = jnp.concatenate([prev[T - d:], cur[:T - d]], axis=0)
        val = jnp.where((i == 0) & (row < d), cur, shifted)
        out_ref[0, :, k * cur.shape[1]:(k + 1) * cur.shape[1]] = val


def kernel(embeddings):
    B, S, D = embeddings.shape
    T = _TC_BLOCK
    out_sc = _sc_part(embeddings)

    grid = (B, S // T)
    return pl.pallas_call(
        _tc_kernel,
        grid=grid,
        in_specs=[
            pl.BlockSpec((1, T, D), lambda b, i: (b, i, 0)),
            pl.BlockSpec((1, T, D), lambda b, i: (b, jnp.maximum(i - 1, 0), 0)),
            pl.BlockSpec(memory_space=pl.ANY),
        ],
        out_specs=pl.BlockSpec((1, T, 3 * D), lambda b, i: (b, i, 0)),
        out_shape=jax.ShapeDtypeStruct((B, S, _K * D), jnp.float32),
        input_output_aliases={2: 0},
    )(embeddings, embeddings, out_sc)


# speedup vs baseline: 51.1193x; 1.0442x over previous
import functools

import jax
import jax.numpy as jnp
from jax import lax
from jax.experimental import pallas as pl
from jax.experimental.pallas import tpu as pltpu
from jax.experimental.pallas import tpu_sc as plsc

_SC_DELAYS = ((3, 8), (4, 16), (5, 32))
_TC_DELAYS = ((0, 1), (1, 2), (2, 4))
_K = 6
_C = 32
_W0 = 64
_W1 = 48
_TC_BLOCK = 256


def _sc_part(embeddings):
    B, S, D = embeddings.shape

    info = plsc.get_sparse_core_info()
    nw = info.num_cores * info.num_subcores
    assert S == 2 * _C * nw
    mesh = plsc.VectorSubcoreMesh(core_axis_name="c", subcore_axis_name="s")

    @functools.partial(
        pl.kernel,
        out_type=jax.ShapeDtypeStruct((B, S, _K * D), jnp.float32),
        mesh=mesh,
        scratch_types=[
            pltpu.VMEM((_W0, D), jnp.float32),
            pltpu.VMEM((_W1, D), jnp.float32),
            pltpu.SemaphoreType.DMA,
            pltpu.SemaphoreType.DMA,
        ],
    )
    def run(emb_hbm, out_hbm, win0, win1, sem0, sem1):
        wid = lax.axis_index("s") * info.num_cores + lax.axis_index("c")
        ta = wid * 2 * _C
        tb = ta + _C

        def copies_a(b, do):
            for k, d in _SC_DELAYS:
                ksl = pl.ds(k * D, D)

                @pl.when(wid > 0)
                def _body(k=k, d=d, ksl=ksl):
                    do(win0.at[pl.ds(_C - d, _C)],
                       out_hbm.at[b, pl.ds(ta, _C), ksl], sem0)

                @pl.when(wid == 0)
                def _first(k=k, d=d, ksl=ksl):
                    if d < _C:
                        do(win0.at[pl.ds(0, d)],
                           out_hbm.at[b, pl.ds(0, d), ksl], sem0)
                        do(win0.at[pl.ds(0, _C - d)],
                           out_hbm.at[b, pl.ds(d, _C - d), ksl], sem0)
                    else:
                        do(win0.at[pl.ds(0, _C)],
                           out_hbm.at[b, pl.ds(0, _C), ksl], sem0)

        def copies_b(b, do):
            for k, d in _SC_DELAYS:
                ksl = pl.ds(k * D, D)
                dst = out_hbm.at[b, pl.ds(tb, _C), ksl]
                if d < 32:
                    do(win1.at[pl.ds(16 - d, _C)], dst, sem1)
                else:
                    @pl.when(wid > 0)
                    def _hi(dst=dst):
                        do(win0.at[pl.ds(_C, _C)], dst, sem0)

                    @pl.when(wid == 0)
                    def _lo(dst=dst):
                        do(win0.at[pl.ds(0, _C)], dst, sem0)

        def issue(src, dst, sem):
            pltpu.async_copy(src, dst, sem)

        def drain(src, dst, sem):
            pltpu.make_async_copy(src, dst, sem).wait()

        for b in range(B):
            if b > 0:
                copies_a(b - 1, drain)
                copies_b(b - 1, lambda s, t, m: drain(s, t, m)
                         if m is sem0 else None)
            @pl.when(wid > 0)
            def _stage_a(b=b):
                pltpu.sync_copy(
                    emb_hbm.at[b, pl.ds(ta - _C, _W0), :], win0)

            @pl.when(wid == 0)
            def _stage_a0(b=b):
                pltpu.sync_copy(
                    emb_hbm.at[b, pl.ds(0, _C), :], win0.at[pl.ds(0, _C)])

            copies_a(b, issue)

            if b > 0:
                copies_b(b - 1, lambda s, t, m: drain(s, t, m)
                         if m is sem1 else None)
            pltpu.sync_copy(emb_hbm.at[b, pl.ds(tb - 16, _W1), :], win1)
            copies_b(b, issue)

        copies_a(B - 1, drain)
        copies_b(B - 1, drain)

    return run(embeddings)


def _tc_kernel(emb_ref, halo_ref, out_sc_ref, out_ref):
    del out_sc_ref
    i = pl.program_id(1)
    T = _TC_BLOCK
    cur = emb_ref[0]
    halo = halo_ref[0]
    row = lax.broadcasted_iota(jnp.int32, (T, 1), 0)
    for k, d in _TC_DELAYS:
        shifted = jnp.concatenate([halo[8 - d:], cur[:T - d]], axis=0)
        val = jnp.where((i == 0) & (row < d), cur, shifted)
        out_ref[0, :, k * cur.shape[1]:(k + 1) * cur.shape[1]] = val


def kernel(embeddings):
    B, S, D = embeddings.shape
    T = _TC_BLOCK
    out_sc = _sc_part(embeddings)

    return pl.pallas_call(
        _tc_kernel,
        grid=(B, S // T),
        in_specs=[
            pl.BlockSpec((1, T, D), lambda b, i: (b, i, 0)),
            pl.BlockSpec((1, 8, D),
                         lambda b, i: (b, jnp.maximum(i * (T // 8) - 1, 0), 0)),
            pl.BlockSpec(memory_space=pl.ANY),
        ],
        out_specs=pl.BlockSpec((1, T, 3 * D), lambda b, i: (b, i, 0)),
        out_shape=jax.ShapeDtypeStruct((B, S, _K * D), jnp.float32),
        input_output_aliases={2: 0},
    )(embeddings, embeddings, out_sc)


# speedup vs baseline: 52.9790x; 1.0364x over previous
import functools

import jax
import jax.numpy as jnp
from jax import lax
from jax.experimental import pallas as pl
from jax.experimental.pallas import tpu as pltpu
from jax.experimental.pallas import tpu_sc as plsc

_SC_DELAYS = ((3, 8), (4, 16), (5, 32))
_TC_DELAYS = ((0, 1), (1, 2), (2, 4))
_K = 6
_HALO = 32
_C = 64
_W = _HALO + _C
_TC_BLOCK = 256


def _sc_part(embeddings):
    B, S, D = embeddings.shape

    info = plsc.get_sparse_core_info()
    nw = info.num_cores * info.num_subcores
    assert S == _C * nw
    mesh = plsc.VectorSubcoreMesh(core_axis_name="c", subcore_axis_name="s")

    @functools.partial(
        pl.kernel,
        out_type=jax.ShapeDtypeStruct((B, S, _K * D), jnp.float32),
        mesh=mesh,
        scratch_types=[
            pltpu.VMEM((_W, D), jnp.float32),
            pltpu.SemaphoreType.DMA,
        ],
    )
    def run(emb_hbm, out_hbm, win, sem):
        cid = lax.axis_index("c")
        sid = lax.axis_index("s")
        wid = sid * info.num_cores + cid
        t0 = wid * _C

        def copies(b, do):
            for k, d in _SC_DELAYS:
                ksl = pl.ds(k * D, D)

                @pl.when(wid > 0)
                def _body(k=k, d=d, ksl=ksl):
                    do(win.at[pl.ds(_HALO - d, _C)],
                       out_hbm.at[b, pl.ds(t0, _C), ksl], sem)

                @pl.when(wid == 0)
                def _first(k=k, d=d, ksl=ksl):
                    do(win.at[pl.ds(0, d)],
                       out_hbm.at[b, pl.ds(0, d), ksl], sem)
                    do(win.at[pl.ds(0, _C - d)],
                       out_hbm.at[b, pl.ds(d, _C - d), ksl], sem)

        def issue(src, dst, sem):
            pltpu.async_copy(src, dst, sem)

        def drain(src, dst, sem):
            pltpu.make_async_copy(src, dst, sem).wait()

        for b in range(B):
            if b >= 1:
                copies(b - 1, drain)

            @pl.when(wid > 0)
            def _stage(b=b):
                pltpu.sync_copy(
                    emb_hbm.at[b, pl.ds(t0 - _HALO, _W), :], win)

            @pl.when(wid == 0)
            def _stage0(b=b):
                pltpu.sync_copy(
                    emb_hbm.at[b, pl.ds(0, _C), :], win.at[pl.ds(0, _C)])

            copies(b, issue)

        copies(B - 1, drain)

    return run(embeddings)


def _tc_kernel(emb_ref, halo_ref, out_sc_ref, out_ref):
    del out_sc_ref
    i = pl.program_id(1)
    T = _TC_BLOCK
    cur = emb_ref[0]
    halo = halo_ref[0]
    row = lax.broadcasted_iota(jnp.int32, (T, 1), 0)
    for k, d in _TC_DELAYS:
        shifted = jnp.concatenate([halo[8 - d:], cur[:T - d]], axis=0)
        val = jnp.where((i == 0) & (row < d), cur, shifted)
        out_ref[0, :, k * cur.shape[1]:(k + 1) * cur.shape[1]] = val


def kernel(embeddings):
    B, S, D = embeddings.shape
    T = _TC_BLOCK
    out_sc = _sc_part(embeddings)

    return pl.pallas_call(
        _tc_kernel,
        grid=(B, S // T),
        in_specs=[
            pl.BlockSpec((1, T, D), lambda b, i: (b, i, 0)),
            pl.BlockSpec((1, 8, D),
                         lambda b, i: (b, jnp.maximum(i * (T // 8) - 1, 0), 0)),
            pl.BlockSpec(memory_space=pl.ANY),
        ],
        out_specs=pl.BlockSpec((1, T, 3 * D), lambda b, i: (b, i, 0)),
        out_shape=jax.ShapeDtypeStruct((B, S, _K * D), jnp.float32),
        input_output_aliases={2: 0},
    )(embeddings, embeddings, out_sc)


# speedup vs baseline: 53.4692x; 1.0093x over previous
import functools

import jax
import jax.numpy as jnp
from jax import lax
from jax.experimental import pallas as pl
from jax.experimental.pallas import tpu as pltpu
from jax.experimental.pallas import tpu_sc as plsc

_SC_DELAYS = ((3, 8), (4, 16), (5, 32))
_TC_DELAYS = ((0, 1), (1, 2), (2, 4))
_K = 6
_COL = 256
_CH = 32
_NCH = _COL // _CH
_TC_BLOCK = 256


def _sc_part(embeddings):
    B, S, D = embeddings.shape

    info = plsc.get_sparse_core_info()
    nw = info.num_cores * info.num_subcores
    ncol = S // _COL
    assert nw == B * ncol
    mesh = plsc.VectorSubcoreMesh(core_axis_name="c", subcore_axis_name="s")

    @functools.partial(
        pl.kernel,
        out_type=jax.ShapeDtypeStruct((B, S, _K * D), jnp.float32),
        mesh=mesh,
        scratch_types=[
            pltpu.VMEM((3, _CH, D), jnp.float32),
            pltpu.SemaphoreType.DMA,
            pltpu.SemaphoreType.DMA,
            pltpu.SemaphoreType.DMA,
        ],
    )
    def run(emb_hbm, out_hbm, ring, sem0, sem1, sem2):
        cid = lax.axis_index("c")
        sid = lax.axis_index("s")
        wid = sid * info.num_cores + cid
        b = wid // ncol
        c = wid % ncol
        t0 = c * _COL
        sems = (sem0, sem1, sem2)

        def win(i):
            return ring.at[(i + 1) % 3]

        def chunk_copies(i, do):
            sem = sems[i % 3]
            r0 = t0 + i * _CH
            for k, d in _SC_DELAYS:
                ksl = pl.ds(k * D, D)
                if i == 0:
                    @pl.when(c > 0)
                    def _body(k=k, d=d, ksl=ksl):
                        if d == _CH:
                            do(win(-1), out_hbm.at[b, pl.ds(r0, _CH), ksl], sem)
                        else:
                            do(win(-1).at[pl.ds(_CH - d, d)],
                               out_hbm.at[b, pl.ds(r0, d), ksl], sem)
                            do(win(0).at[pl.ds(0, _CH - d)],
                               out_hbm.at[b, pl.ds(r0 + d, _CH - d), ksl], sem)

                    @pl.when(c == 0)
                    def _head(k=k, d=d, ksl=ksl):
                        if d == _CH:
                            do(win(0), out_hbm.at[b, pl.ds(0, _CH), ksl], sem)
                        else:
                            do(win(0).at[pl.ds(0, d)],
                               out_hbm.at[b, pl.ds(0, d), ksl], sem)
                            do(win(0).at[pl.ds(0, _CH - d)],
                               out_hbm.at[b, pl.ds(d, _CH - d), ksl], sem)
                else:
                    if d == _CH:
                        do(win(i - 1), out_hbm.at[b, pl.ds(r0, _CH), ksl], sem)
                    else:
                        do(win(i - 1).at[pl.ds(_CH - d, d)],
                           out_hbm.at[b, pl.ds(r0, d), ksl], sem)
                        do(win(i).at[pl.ds(0, _CH - d)],
                           out_hbm.at[b, pl.ds(r0 + d, _CH - d), ksl], sem)

        def issue(src, dst, sem):
            pltpu.async_copy(src, dst, sem)

        def drain(src, dst, sem):
            pltpu.make_async_copy(src, dst, sem).wait()

        @pl.when(c > 0)
        def _halo():
            pltpu.sync_copy(emb_hbm.at[b, pl.ds(t0 - _CH, _CH), :], win(-1))

        pltpu.sync_copy(emb_hbm.at[b, pl.ds(t0, _CH), :], win(0))
        chunk_copies(0, issue)
        for i in range(1, _NCH):
            if i >= 2:
                chunk_copies(i - 2, drain)
            pltpu.sync_copy(
                emb_hbm.at[b, pl.ds(t0 + i * _CH, _CH), :], win(i))
            chunk_copies(i, issue)

        chunk_copies(_NCH - 2, drain)
        chunk_copies(_NCH - 1, drain)

    return run(embeddings)


def _tc_kernel(emb_ref, halo_ref, out_sc_ref, out_ref):
    del out_sc_ref
    i = pl.program_id(1)
    T = _TC_BLOCK
    cur = emb_ref[0]
    halo = halo_ref[0]
    row = lax.broadcasted_iota(jnp.int32, (T, 1), 0)
    for k, d in _TC_DELAYS:
        shifted = jnp.concatenate([halo[8 - d:], cur[:T - d]], axis=0)
        val = jnp.where((i == 0) & (row < d), cur, shifted)
        out_ref[0, :, k * cur.shape[1]:(k + 1) * cur.shape[1]] = val


def kernel(embeddings):
    B, S, D = embeddings.shape
    T = _TC_BLOCK
    out_sc = _sc_part(embeddings)

    return pl.pallas_call(
        _tc_kernel,
        grid=(B, S // T),
        in_specs=[
            pl.BlockSpec((1, T, D), lambda b, i: (b, i, 0)),
            pl.BlockSpec((1, 8, D),
                         lambda b, i: (b, jnp.maximum(i * (T // 8) - 1, 0), 0)),
            pl.BlockSpec(memory_space=pl.ANY),
        ],
        out_specs=pl.BlockSpec((1, T, 3 * D), lambda b, i: (b, i, 0)),
        out_shape=jax.ShapeDtypeStruct((B, S, _K * D), jnp.float32),
        input_output_aliases={2: 0},
    )(embeddings, embeddings, out_sc)


# speedup vs baseline: 55.4660x; 1.0373x over previous
import functools

import jax
import jax.numpy as jnp
from jax import lax
from jax.experimental import pallas as pl
from jax.experimental.pallas import tpu as pltpu
from jax.experimental.pallas import tpu_sc as plsc

_SC_DELAYS = ((3, 8), (4, 16), (5, 32))
_TC_DELAYS = ((0, 1), (1, 2), (2, 4))
_K = 6
_COL = 256
_CH = 32
_NCH = _COL // _CH
_TC_BLOCK = 512


def _sc_part(embeddings):
    B, S, D = embeddings.shape

    info = plsc.get_sparse_core_info()
    nw = info.num_cores * info.num_subcores
    ncol = S // _COL
    assert nw == B * ncol
    mesh = plsc.VectorSubcoreMesh(core_axis_name="c", subcore_axis_name="s")

    @functools.partial(
        pl.kernel,
        out_type=jax.ShapeDtypeStruct((B, S, _K * D), jnp.float32),
        mesh=mesh,
        scratch_types=[
            pltpu.VMEM((3, _CH, D), jnp.float32),
            pltpu.SemaphoreType.DMA,
            pltpu.SemaphoreType.DMA,
            pltpu.SemaphoreType.DMA,
        ],
    )
    def run(emb_hbm, out_hbm, ring, sem0, sem1, sem2):
        cid = lax.axis_index("c")
        sid = lax.axis_index("s")
        wid = sid * info.num_cores + cid
        b = wid // ncol
        c = wid % ncol
        t0 = c * _COL
        sems = (sem0, sem1, sem2)

        def win(i):
            return ring.at[(i + 1) % 3]

        def chunk_copies(i, do):
            sem = sems[i % 3]
            r0 = t0 + i * _CH
            for k, d in _SC_DELAYS:
                ksl = pl.ds(k * D, D)
                if i == 0:
                    @pl.when(c > 0)
                    def _body(k=k, d=d, ksl=ksl):
                        if d == _CH:
                            do(win(-1), out_hbm.at[b, pl.ds(r0, _CH), ksl], sem)
                        else:
                            do(win(-1).at[pl.ds(_CH - d, d)],
                               out_hbm.at[b, pl.ds(r0, d), ksl], sem)
                            do(win(0).at[pl.ds(0, _CH - d)],
                               out_hbm.at[b, pl.ds(r0 + d, _CH - d), ksl], sem)

                    @pl.when(c == 0)
                    def _head(k=k, d=d, ksl=ksl):
                        if d == _CH:
                            do(win(0), out_hbm.at[b, pl.ds(0, _CH), ksl], sem)
                        else:
                            do(win(0).at[pl.ds(0, d)],
                               out_hbm.at[b, pl.ds(0, d), ksl], sem)
                            do(win(0).at[pl.ds(0, _CH - d)],
                               out_hbm.at[b, pl.ds(d, _CH - d), ksl], sem)
                else:
                    if d == _CH:
                        do(win(i - 1), out_hbm.at[b, pl.ds(r0, _CH), ksl], sem)
                    else:
                        do(win(i - 1).at[pl.ds(_CH - d, d)],
                           out_hbm.at[b, pl.ds(r0, d), ksl], sem)
                        do(win(i).at[pl.ds(0, _CH - d)],
                           out_hbm.at[b, pl.ds(r0 + d, _CH - d), ksl], sem)

        def issue(src, dst, sem):
            pltpu.async_copy(src, dst, sem)

        def drain(src, dst, sem):
            pltpu.make_async_copy(src, dst, sem).wait()

        @pl.when(c > 0)
        def _halo():
            pltpu.sync_copy(emb_hbm.at[b, pl.ds(t0 - _CH, _CH), :], win(-1))

        pltpu.sync_copy(emb_hbm.at[b, pl.ds(t0, _CH), :], win(0))
        chunk_copies(0, issue)
        for i in range(1, _NCH):
            if i >= 2:
                chunk_copies(i - 2, drain)
            pltpu.sync_copy(
                emb_hbm.at[b, pl.ds(t0 + i * _CH, _CH), :], win(i))
            chunk_copies(i, issue)

        chunk_copies(_NCH - 2, drain)
        chunk_copies(_NCH - 1, drain)

    return run(embeddings)


def _tc_kernel(emb_ref, halo_ref, out_sc_ref, out_ref):
    del out_sc_ref
    i = pl.program_id(1)
    T = _TC_BLOCK
    cur = emb_ref[0]
    halo = halo_ref[0]
    row = lax.broadcasted_iota(jnp.int32, (T, 1), 0)
    for k, d in _TC_DELAYS:
        shifted = jnp.concatenate([halo[8 - d:], cur[:T - d]], axis=0)
        val = jnp.where((i == 0) & (row < d), cur, shifted)
        out_ref[0, :, k * cur.shape[1]:(k + 1) * cur.shape[1]] = val


def kernel(embeddings):
    B, S, D = embeddings.shape
    T = _TC_BLOCK
    out_sc = _sc_part(embeddings)

    return pl.pallas_call(
        _tc_kernel,
        grid=(B, S // T),
        in_specs=[
            pl.BlockSpec((1, T, D), lambda b, i: (b, i, 0)),
            pl.BlockSpec((1, 8, D),
                         lambda b, i: (b, jnp.maximum(i * (T // 8) - 1, 0), 0)),
            pl.BlockSpec(memory_space=pl.ANY),
        ],
        out_specs=pl.BlockSpec((1, T, 3 * D), lambda b, i: (b, i, 0)),
        out_shape=jax.ShapeDtypeStruct((B, S, _K * D), jnp.float32),
        input_output_aliases={2: 0},
    )(embeddings, embeddings, out_sc)
